# baseline (speedup 1.0000x reference)
import jax
import jax.numpy as jnp
from jax.experimental import pallas as pl

N = 10000
E = 160000
WN = 32
WK = 64
DEPTH = 4


def _edge_mlp_body(ea_ref, k1w_ref, k1b_ref, k2w_ref, k2b_ref, k3w_ref, k3b_ref,
                   out_ref):
    ea = ea_ref[...]
    e1 = jnp.maximum(jnp.dot(ea, k1w_ref[...], preferred_element_type=jnp.float32)
                     + k1b_ref[...], 0.0)
    e2 = jnp.maximum(jnp.dot(e1, k2w_ref[...], preferred_element_type=jnp.float32)
                     + k2b_ref[...], 0.0)
    out_ref[...] = (jnp.dot(e2, k3w_ref[...], preferred_element_type=jnp.float32)
                    + k3b_ref[...])


def kernel(x, edge_index, edge_attr, fc1_w, fc1_b, k1_w, k1_b, k2_w, k2_b,
           k3_w, k3_b, root, conv_bias, fc2_w, fc2_b, fc3_w, fc3_b):
    src = edge_index[0]
    dst = edge_index[1]
    h = x @ fc1_w + fc1_b

    B = 800
    grid = E // B
    kern_flat = pl.pallas_call(
        _edge_mlp_body,
        grid=(grid,),
        in_specs=[
            pl.BlockSpec((B, 4), lambda i: (i, 0)),
            pl.BlockSpec((4, WK // 2), lambda i: (0, 0)),
            pl.BlockSpec((WK // 2,), lambda i: (0,)),
            pl.BlockSpec((WK // 2, WK), lambda i: (0, 0)),
            pl.BlockSpec((WK,), lambda i: (0,)),
            pl.BlockSpec((WK, WN * WN), lambda i: (0, 0)),
            pl.BlockSpec((WN * WN,), lambda i: (0,)),
        ],
        out_specs=pl.BlockSpec((B, WN * WN), lambda i: (i, 0)),
        out_shape=jax.ShapeDtypeStruct((E, WN * WN), jnp.float32),
    )(edge_attr, k1_w, k1_b, k2_w, k2_b, k3_w, k3_b)
    kern = kern_flat.reshape(E, WN, WN)

    deg = jax.ops.segment_sum(jnp.ones((E,), jnp.float32), dst, num_segments=N)
    deg = jnp.clip(deg, 1.0)[:, None]
    for d in range(DEPTH):
        msg = jnp.einsum('ei,eio->eo', h[src], kern)
        agg = jax.ops.segment_sum(msg, dst, num_segments=N) / deg
        h = agg + h @ root + conv_bias
        if d != DEPTH - 1:
            h = jax.nn.relu(h)
    h = jax.nn.relu(h @ fc2_w + fc2_b)
    out = h @ fc3_w + fc3_b
    return out
